# baseline (speedup 1.0000x reference)
import jax
import jax.numpy as jnp
from jax import lax
from jax.experimental import pallas as pl
from jax.experimental.pallas import tpu as pltpu
from jax.experimental.pallas import tpu_sc as plsc

N = 10000
E = 320000
D = 128
DE = 16

NC = 2
NS = 16
NW = NC * NS
C = 64
NCHUNKS = E // C
NPAD = 10240

RB = 2000
NBB = E // RB
NAB = N // RB


NH = 5120
NHP = 5248
TRASH = 5200
ZROWS = NHP // NS
DROWS = NH // NS


def _remap(dst_v, idxp_v, lo):
    for u in range(C // 16):
        su = pl.ds(u * 16, 16)
        t = dst_v[su] - lo
        ok = (t >= 0) & (t < NH)
        idxp_v[su] = jnp.where(ok, t, TRASH)


def _zero_acc(zeros_hbm, agg_sp, sid):
    r0 = sid * ZROWS
    for d in range(ZROWS // C):
        pltpu.sync_copy(zeros_hbm, agg_sp.at[pl.ds(r0 + d * C, C)])
    rem = ZROWS % C
    if rem:
        pltpu.sync_copy(zeros_hbm.at[pl.ds(0, rem)],
                        agg_sp.at[pl.ds(r0 + (ZROWS // C) * C, rem)])


def _dump_acc(agg_sp, agg_hbm, cid, sid, out_base):
    for d in range(DROWS // C):
        r0 = sid * DROWS + d * C
        pltpu.sync_copy(agg_sp.at[pl.ds(r0, C)],
                        agg_hbm.at[cid, pl.ds(out_base + r0, C)])


def _edge_sc(y_hbm, z_hbm, src_hbm, dst_hbm, zeros_hbm, msg_hbm, agg_hbm,
             src_v, dst_v, idxp_v, yg_v, z_v, m_v, agg_sp, sem):
    cid = lax.axis_index("c")
    sid = lax.axis_index("s")
    wid = sid * NC + cid
    _zero_acc(zeros_hbm, agg_sp, sid)
    plsc.subcore_barrier()
    nchunk = jnp.where(wid < 8, 157, 156)
    base0 = C * (156 * wid + jnp.minimum(wid, 8))

    @pl.loop(0, nchunk)
    def _chunk(k):
        base = base0 + k * C
        pltpu.sync_copy(src_hbm.at[pl.ds(base, C)], src_v)
        pltpu.sync_copy(dst_hbm.at[pl.ds(base, C)], dst_v)
        pltpu.async_copy(y_hbm.at[src_v], yg_v, sem).wait()
        pltpu.sync_copy(z_hbm.at[pl.ds(pl.multiple_of(base // 8, 8), C // 8)],
                        z_v)

        @pl.loop(0, C)
        def _row(r):
            zr = r // 8
            zc = (r % 8) * D
            for j in range(D // 16):
                s = pl.ds(j * 16, 16)
                m_v[r, s] = jnp.maximum(
                    yg_v[r, s] + z_v[zr, pl.ds(zc + j * 16, 16)], 0.0)

        pltpu.sync_copy(m_v, msg_hbm.at[pl.ds(base, C)])
        _remap(dst_v, idxp_v, 0)
        pltpu.sync_copy(m_v, agg_sp.at[idxp_v], add=True)

    plsc.subcore_barrier()
    _dump_acc(agg_sp, agg_hbm, cid, sid, 0)
    plsc.subcore_barrier()
    _zero_acc(zeros_hbm, agg_sp, sid)
    plsc.subcore_barrier()

    @pl.loop(0, nchunk)
    def _chunk2(k):
        base = base0 + k * C
        pltpu.sync_copy(dst_hbm.at[pl.ds(base, C)], dst_v)
        pltpu.sync_copy(msg_hbm.at[pl.ds(base, C)], m_v)
        _remap(dst_v, idxp_v, NH)
        pltpu.sync_copy(m_v, agg_sp.at[idxp_v], add=True)

    plsc.subcore_barrier()
    _dump_acc(agg_sp, agg_hbm, cid, sid, NH)


def _edge_phase(y, z, src_r, dst_r, zeros):
    mesh = plsc.VectorSubcoreMesh(core_axis_name="c", subcore_axis_name="s")
    return pl.kernel(
        _edge_sc,
        out_type=(
            jax.ShapeDtypeStruct((E, D), jnp.float32),
            jax.ShapeDtypeStruct((NC, NPAD, D), jnp.float32),
        ),
        mesh=mesh,
        scratch_types=[
            pltpu.VMEM((C,), jnp.int32),
            pltpu.VMEM((C,), jnp.int32),
            pltpu.VMEM((C,), jnp.int32),
            pltpu.VMEM((C, D), jnp.float32),
            pltpu.VMEM((C // 8, 8 * D), jnp.float32),
            pltpu.VMEM((C, D), jnp.float32),
            pltpu.VMEM_SHARED((NHP, D), jnp.float32),
            pltpu.SemaphoreType.DMA,
        ],
    )(y, z, src_r, dst_r, zeros)


def _prep_y(x, W1):
    def body(x_ref, w_ref, y_ref):
        y_ref[...] = jnp.dot(x_ref[...], w_ref[...],
                             preferred_element_type=jnp.float32)

    return pl.pallas_call(
        body,
        out_shape=jax.ShapeDtypeStruct((N, D), jnp.float32),
    )(x, W1)


def _prep_z(ea8, W2b):
    def body(ea_ref, w_ref, z_ref):
        z_ref[...] = jnp.dot(ea_ref[...], w_ref[...],
                             preferred_element_type=jnp.float32)

    zrb = 400
    return pl.pallas_call(
        body,
        grid=(E // 8 // zrb,),
        in_specs=[
            pl.BlockSpec((zrb, 8 * DE), lambda i: (i, 0)),
            pl.BlockSpec((8 * DE, 8 * D), lambda i: (0, 0)),
        ],
        out_specs=pl.BlockSpec((zrb, 8 * D), lambda i: (i, 0)),
        out_shape=jax.ShapeDtypeStruct((E // 8, 8 * D), jnp.float32),
    )(ea8, W2b)


def _atoms_phase(agg2, x, W_node, W_mol_a, W_mol_b, W_ffn, b_ffn_2d):
    def body(agg_ref, x_ref, wn_ref, wa_ref, wb_ref, wf_ref,
             bf_ref, atoms_ref, mv_ref, c_ref):
        agg = agg_ref[0, :N, :] + agg_ref[1, :N, :]
        pre = jnp.dot(agg, wn_ref[...],
                      preferred_element_type=jnp.float32) + x_ref[...]
        atoms = jnp.maximum(pre, 0.0)
        atoms_ref[...] = atoms
        mean_a = jnp.sum(atoms, axis=0, keepdims=True) * (1.0 / N)
        mean_b = jnp.sum(agg, axis=0, keepdims=True) * (1.0 / E)
        mva = jnp.dot(mean_a, wa_ref[...], preferred_element_type=jnp.float32)
        mvb = jnp.dot(mean_b, wb_ref[...], preferred_element_type=jnp.float32)
        mv_ref[0:1, :] = mvb
        mv_ref[1:2, :] = mva
        w2 = wf_ref[D:, :]
        c_ref[0:1, :] = jnp.dot(mvb, w2,
                                preferred_element_type=jnp.float32) + bf_ref[...]
        c_ref[1:2, :] = jnp.dot(mva, w2,
                                preferred_element_type=jnp.float32) + bf_ref[...]

    return pl.pallas_call(
        body,
        out_shape=(
            jax.ShapeDtypeStruct((N, D), jnp.float32),
            jax.ShapeDtypeStruct((2, D), jnp.float32),
            jax.ShapeDtypeStruct((2, 1), jnp.float32),
        ),
    )(agg2, x, W_node, W_mol_a, W_mol_b, W_ffn, b_ffn_2d)


def _fill_phase(msg, atoms_v, mv, c, w1):
    def body(msg_ref, at_ref, mv_ref, c_ref, w1_ref,
             v_ref, r_ref, t_ref):
        i = pl.program_id(0)
        is_bond = i < NBB
        blk = jnp.where(is_bond, msg_ref[...], at_ref[...])
        mvrow = jnp.where(is_bond, mv_ref[0:1, :], mv_ref[1:2, :])
        cc = jnp.where(is_bond, c_ref[0:1, :], c_ref[1:2, :])
        v_ref[:, :D] = blk
        v_ref[:, D:] = jnp.broadcast_to(mvrow, (RB, D))
        t_ref[...] = jnp.sum(blk, axis=1, keepdims=True).reshape(1, 1, RB)
        rr = jnp.dot(blk, w1_ref[...], preferred_element_type=jnp.float32)
        r_ref[...] = rr.reshape(1, 1, RB) + cc

    return pl.pallas_call(
        body,
        grid=(NBB + NAB,),
        in_specs=[
            pl.BlockSpec((RB, D), lambda i: (jnp.minimum(i, NBB - 1), 0)),
            pl.BlockSpec((RB, D), lambda i: (jnp.maximum(i - NBB, 0), 0)),
            pl.BlockSpec((2, D), lambda i: (0, 0)),
            pl.BlockSpec((2, 1), lambda i: (0, 0)),
            pl.BlockSpec((D, 1), lambda i: (0, 0)),
        ],
        out_specs=[
            pl.BlockSpec((RB, 2 * D), lambda i: (i, 0)),
            pl.BlockSpec((1, 1, RB), lambda i: (i, 0, 0)),
            pl.BlockSpec((1, 1, RB), lambda i: (i, 0, 0)),
        ],
        out_shape=[
            jax.ShapeDtypeStruct((E + N, 2 * D), jnp.float32),
            jax.ShapeDtypeStruct((NBB + NAB, 1, RB), jnp.float32),
            jax.ShapeDtypeStruct((NBB + NAB, 1, RB), jnp.float32),
        ],
    )(msg, atoms_v, mv, c, w1)


def kernel(x, edge_index, edge_attr, W_msg, W_node, W_mol_a, W_mol_b, W_ffn,
           b_ffn):
    x = x.astype(jnp.float32)
    ei = edge_index.astype(jnp.int32)
    src_r = ei[0]
    dst_r = ei[1]
    W1 = W_msg[:D]
    W2 = W_msg[D:]
    y = _prep_y(x, W1)
    ea8 = edge_attr.reshape(E // 8, 8 * DE)
    W2b = jnp.kron(jnp.eye(8, dtype=jnp.float32), W2)
    z = _prep_z(ea8, W2b)
    zeros = jnp.zeros((C, D), jnp.float32)
    msg, agg2 = _edge_phase(y, z, src_r, dst_r, zeros)
    atoms_v, mv, c = _atoms_phase(agg2, x, W_node, W_mol_a, W_mol_b,
                                  W_ffn, jnp.reshape(b_ffn, (1, 1)))
    w1 = W_ffn[:D]
    v_all, r2, t2 = _fill_phase(msg, atoms_v, mv, c, w1)
    r_all = r2.reshape(E + N, 1)
    t_full = t2.reshape(E + N, 1)
    t_all = jnp.concatenate([t_full[:E], t_full[E + 1:]], axis=0)
    return (r_all, t_all, v_all)


# speedup vs baseline: 1.3349x; 1.3349x over previous
import jax
import jax.numpy as jnp
from jax import lax
from jax.experimental import pallas as pl
from jax.experimental.pallas import tpu as pltpu
from jax.experimental.pallas import tpu_sc as plsc

N = 10000
E = 320000
D = 128
DE = 16

NC = 2
NS = 16
NW = NC * NS
C = 64
NCHUNKS = E // C
NPAD = 10240

RB = 2000
NBB = E // RB
NAB = N // RB


NH = 5120
NHP = 5248
TRASH = 5200
ZROWS = NHP // NS
DROWS = NH // NS


def _remap(dst_v, idxp_v, lo):
    for u in range(C // 16):
        su = pl.ds(u * 16, 16)
        t = dst_v[su] - lo
        ok = (t >= 0) & (t < NH)
        idxp_v[su] = jnp.where(ok, t, TRASH)


def _zero_acc(zeros_hbm, agg_sp, sid):
    r0 = sid * ZROWS
    for d in range(ZROWS // C):
        pltpu.sync_copy(zeros_hbm, agg_sp.at[pl.ds(r0 + d * C, C)])
    rem = ZROWS % C
    if rem:
        pltpu.sync_copy(zeros_hbm.at[pl.ds(0, rem)],
                        agg_sp.at[pl.ds(r0 + (ZROWS // C) * C, rem)])


def _dump_acc(agg_sp, agg_hbm, cid, sid, out_base):
    for d in range(DROWS // C):
        r0 = sid * DROWS + d * C
        pltpu.sync_copy(agg_sp.at[pl.ds(r0, C)],
                        agg_hbm.at[cid, pl.ds(out_base + r0, C)])


def _edge_sc(y_hbm, z_hbm, src_hbm, dst_hbm, zeros_hbm, msg_hbm, agg_hbm,
             src0, src1, dst0, dst1, ixp0, ixp1, yg0, yg1, z0, z1, m0, m1,
             agg_sp,
             ss0, ss1, sd0, sd1, sz0, sz1, sg0, sg1, sw0, sw1, sc0, sc1):
    cid = lax.axis_index("c")
    sid = lax.axis_index("s")
    wid = sid * NC + cid
    src_v = (src0, src1)
    dst_v = (dst0, dst1)
    ixp_v = (ixp0, ixp1)
    yg_v = (yg0, yg1)
    z_v = (z0, z1)
    m_v = (m0, m1)
    s_src = (ss0, ss1)
    s_dst = (sd0, sd1)
    s_z = (sz0, sz1)
    s_g = (sg0, sg1)
    s_w = (sw0, sw1)
    s_sc = (sc0, sc1)

    npair = jnp.where(wid < 4, 79, 78)
    base0 = 2 * C * (78 * wid + jnp.minimum(wid, 4))

    def ebase(k):
        return base0 + k * C

    def zslice(k):
        return z_hbm.at[pl.ds(pl.multiple_of(ebase(k) // 8, 8), C // 8)]

    def issue_pref(k, b):
        pltpu.async_copy(src_hbm.at[pl.ds(ebase(k), C)], src_v[b], s_src[b])
        pltpu.async_copy(dst_hbm.at[pl.ds(ebase(k), C)], dst_v[b], s_dst[b])
        pltpu.async_copy(zslice(k), z_v[b], s_z[b])

    def wait_pref(k, b):
        pltpu.make_async_copy(src_hbm.at[pl.ds(ebase(k), C)], src_v[b],
                              s_src[b]).wait()
        pltpu.make_async_copy(dst_hbm.at[pl.ds(ebase(k), C)], dst_v[b],
                              s_dst[b]).wait()
        pltpu.make_async_copy(zslice(k), z_v[b], s_z[b]).wait()

    def issue_gather(k, b):
        pltpu.async_copy(y_hbm.at[src_v[b]], yg_v[b], s_g[b])

    def wait_gather(b):
        pltpu.make_async_copy(y_hbm.at[src_v[b]], yg_v[b], s_g[b]).wait()

    def compute(b):
        @pl.loop(0, C)
        def _row(r):
            zr = r // 8
            zc = (r % 8) * D
            for j in range(D // 16):
                s = pl.ds(j * 16, 16)
                m_v[b][r, s] = jnp.maximum(
                    yg_v[b][r, s] + z_v[b][zr, pl.ds(zc + j * 16, 16)], 0.0)

    _zero_acc(zeros_hbm, agg_sp, sid)
    plsc.subcore_barrier()

    issue_pref(0, 0)
    issue_pref(1, 1)
    wait_pref(0, 0)
    issue_gather(0, 0)

    @pl.loop(0, npair)
    def _pair(g):
        for b in (0, 1):
            k = 2 * g + b
            nb = 1 - b
            if b == 0:
                wait_pref(k + 1, nb)
                issue_gather(k + 1, nb)
            else:
                @pl.when(g < npair - 1)
                def _():
                    wait_pref(k + 1, nb)
                    issue_gather(k + 1, nb)
            wait_gather(b)

            @pl.when(g >= 1)
            def _():
                pltpu.make_async_copy(m_v[b],
                                      msg_hbm.at[pl.ds(ebase(k - 2), C)],
                                      s_w[b]).wait()
                pltpu.make_async_copy(m_v[b], agg_sp.at[ixp_v[b]],
                                      s_sc[b]).wait()

            compute(b)
            _remap(dst_v[b], ixp_v[b], 0)
            pltpu.async_copy(m_v[b], msg_hbm.at[pl.ds(ebase(k), C)], s_w[b])
            pltpu.async_copy(m_v[b], agg_sp.at[ixp_v[b]], s_sc[b], add=True)

            @pl.when(g < npair - 1)
            def _():
                issue_pref(k + 2, b)

    for b in (0, 1):
        pltpu.make_async_copy(m_v[b], msg_hbm.at[pl.ds(0, C)], s_w[b]).wait()
        pltpu.make_async_copy(m_v[b], agg_sp.at[ixp_v[b]], s_sc[b]).wait()

    plsc.subcore_barrier()
    _dump_acc(agg_sp, agg_hbm, cid, sid, 0)
    plsc.subcore_barrier()
    _zero_acc(zeros_hbm, agg_sp, sid)
    plsc.subcore_barrier()

    def issue_pref2(k, b):
        pltpu.async_copy(dst_hbm.at[pl.ds(ebase(k), C)], dst_v[b], s_dst[b])
        pltpu.async_copy(msg_hbm.at[pl.ds(ebase(k), C)], m_v[b], s_w[b])

    def wait_pref2(k, b):
        pltpu.make_async_copy(dst_hbm.at[pl.ds(ebase(k), C)], dst_v[b],
                              s_dst[b]).wait()
        pltpu.make_async_copy(msg_hbm.at[pl.ds(ebase(k), C)], m_v[b],
                              s_w[b]).wait()

    issue_pref2(0, 0)

    @pl.loop(0, npair)
    def _pair2(g):
        for b in (0, 1):
            k = 2 * g + b
            nb = 1 - b
            wait_pref2(k, b)
            _remap(dst_v[b], ixp_v[b], NH)
            pltpu.async_copy(m_v[b], agg_sp.at[ixp_v[b]], s_sc[b], add=True)
            if b == 0:
                @pl.when(g >= 1)
                def _():
                    pltpu.make_async_copy(m_v[nb], agg_sp.at[ixp_v[nb]],
                                          s_sc[nb]).wait()
                issue_pref2(k + 1, nb)
            else:
                @pl.when(g < npair - 1)
                def _():
                    pltpu.make_async_copy(m_v[nb], agg_sp.at[ixp_v[nb]],
                                          s_sc[nb]).wait()
                    issue_pref2(k + 1, nb)

    for b in (0, 1):
        pltpu.make_async_copy(m_v[b], agg_sp.at[ixp_v[b]], s_sc[b]).wait()

    plsc.subcore_barrier()
    _dump_acc(agg_sp, agg_hbm, cid, sid, NH)


def _edge_phase(y, z, src_r, dst_r, zeros):
    mesh = plsc.VectorSubcoreMesh(core_axis_name="c", subcore_axis_name="s")
    return pl.kernel(
        _edge_sc,
        out_type=(
            jax.ShapeDtypeStruct((E, D), jnp.float32),
            jax.ShapeDtypeStruct((NC, NPAD, D), jnp.float32),
        ),
        mesh=mesh,
        scratch_types=(
            [pltpu.VMEM((C,), jnp.int32)] * 6 +
            [pltpu.VMEM((C, D), jnp.float32)] * 2 +
            [pltpu.VMEM((C // 8, 8 * D), jnp.float32)] * 2 +
            [pltpu.VMEM((C, D), jnp.float32)] * 2 +
            [pltpu.VMEM_SHARED((NHP, D), jnp.float32)] +
            [pltpu.SemaphoreType.DMA] * 12
        ),
    )(y, z, src_r, dst_r, zeros)


def _prep_y(x, W1):
    def body(x_ref, w_ref, y_ref):
        y_ref[...] = jnp.dot(x_ref[...], w_ref[...],
                             preferred_element_type=jnp.float32)

    return pl.pallas_call(
        body,
        out_shape=jax.ShapeDtypeStruct((N, D), jnp.float32),
    )(x, W1)


def _prep_z(ea8, W2b):
    def body(ea_ref, w_ref, z_ref):
        z_ref[...] = jnp.dot(ea_ref[...], w_ref[...],
                             preferred_element_type=jnp.float32)

    zrb = 400
    return pl.pallas_call(
        body,
        grid=(E // 8 // zrb,),
        in_specs=[
            pl.BlockSpec((zrb, 8 * DE), lambda i: (i, 0)),
            pl.BlockSpec((8 * DE, 8 * D), lambda i: (0, 0)),
        ],
        out_specs=pl.BlockSpec((zrb, 8 * D), lambda i: (i, 0)),
        out_shape=jax.ShapeDtypeStruct((E // 8, 8 * D), jnp.float32),
    )(ea8, W2b)


def _atoms_phase(agg2, x, W_node, W_mol_a, W_mol_b, W_ffn, b_ffn_2d):
    def body(agg_ref, x_ref, wn_ref, wa_ref, wb_ref, wf_ref,
             bf_ref, atoms_ref, mv_ref, c_ref):
        agg = agg_ref[0, :N, :] + agg_ref[1, :N, :]
        pre = jnp.dot(agg, wn_ref[...],
                      preferred_element_type=jnp.float32) + x_ref[...]
        atoms = jnp.maximum(pre, 0.0)
        atoms_ref[...] = atoms
        mean_a = jnp.sum(atoms, axis=0, keepdims=True) * (1.0 / N)
        mean_b = jnp.sum(agg, axis=0, keepdims=True) * (1.0 / E)
        mva = jnp.dot(mean_a, wa_ref[...], preferred_element_type=jnp.float32)
        mvb = jnp.dot(mean_b, wb_ref[...], preferred_element_type=jnp.float32)
        mv_ref[0:1, :] = mvb
        mv_ref[1:2, :] = mva
        w2 = wf_ref[D:, :]
        c_ref[0:1, :] = jnp.dot(mvb, w2,
                                preferred_element_type=jnp.float32) + bf_ref[...]
        c_ref[1:2, :] = jnp.dot(mva, w2,
                                preferred_element_type=jnp.float32) + bf_ref[...]

    return pl.pallas_call(
        body,
        out_shape=(
            jax.ShapeDtypeStruct((N, D), jnp.float32),
            jax.ShapeDtypeStruct((2, D), jnp.float32),
            jax.ShapeDtypeStruct((2, 1), jnp.float32),
        ),
    )(agg2, x, W_node, W_mol_a, W_mol_b, W_ffn, b_ffn_2d)


def _fill_phase(msg, atoms_v, mv, c, w1):
    def body(msg_ref, at_ref, mv_ref, c_ref, w1_ref,
             v_ref, r_ref, t_ref):
        i = pl.program_id(0)
        is_bond = i < NBB
        blk = jnp.where(is_bond, msg_ref[...], at_ref[...])
        mvrow = jnp.where(is_bond, mv_ref[0:1, :], mv_ref[1:2, :])
        cc = jnp.where(is_bond, c_ref[0:1, :], c_ref[1:2, :])
        v_ref[:, :D] = blk
        v_ref[:, D:] = jnp.broadcast_to(mvrow, (RB, D))
        t_ref[...] = jnp.sum(blk, axis=1, keepdims=True).reshape(1, 1, RB)
        rr = jnp.dot(blk, w1_ref[...], preferred_element_type=jnp.float32)
        r_ref[...] = rr.reshape(1, 1, RB) + cc

    return pl.pallas_call(
        body,
        grid=(NBB + NAB,),
        in_specs=[
            pl.BlockSpec((RB, D), lambda i: (jnp.minimum(i, NBB - 1), 0)),
            pl.BlockSpec((RB, D), lambda i: (jnp.maximum(i - NBB, 0), 0)),
            pl.BlockSpec((2, D), lambda i: (0, 0)),
            pl.BlockSpec((2, 1), lambda i: (0, 0)),
            pl.BlockSpec((D, 1), lambda i: (0, 0)),
        ],
        out_specs=[
            pl.BlockSpec((RB, 2 * D), lambda i: (i, 0)),
            pl.BlockSpec((1, 1, RB), lambda i: (i, 0, 0)),
            pl.BlockSpec((1, 1, RB), lambda i: (i, 0, 0)),
        ],
        out_shape=[
            jax.ShapeDtypeStruct((E + N, 2 * D), jnp.float32),
            jax.ShapeDtypeStruct((NBB + NAB, 1, RB), jnp.float32),
            jax.ShapeDtypeStruct((NBB + NAB, 1, RB), jnp.float32),
        ],
    )(msg, atoms_v, mv, c, w1)


def kernel(x, edge_index, edge_attr, W_msg, W_node, W_mol_a, W_mol_b, W_ffn,
           b_ffn):
    x = x.astype(jnp.float32)
    ei = edge_index.astype(jnp.int32)
    src_r = ei[0]
    dst_r = ei[1]
    W1 = W_msg[:D]
    W2 = W_msg[D:]
    y = _prep_y(x, W1)
    ea8 = edge_attr.reshape(E // 8, 8 * DE)
    W2b = jnp.kron(jnp.eye(8, dtype=jnp.float32), W2)
    z = _prep_z(ea8, W2b)
    zeros = jnp.zeros((C, D), jnp.float32)
    msg, agg2 = _edge_phase(y, z, src_r, dst_r, zeros)
    atoms_v, mv, c = _atoms_phase(agg2, x, W_node, W_mol_a, W_mol_b,
                                  W_ffn, jnp.reshape(b_ffn, (1, 1)))
    w1 = W_ffn[:D]
    v_all, r2, t2 = _fill_phase(msg, atoms_v, mv, c, w1)
    r_all = r2.reshape(E + N, 1)
    t_full = t2.reshape(E + N, 1)
    t_all = jnp.concatenate([t_full[:E], t_full[E + 1:]], axis=0)
    return (r_all, t_all, v_all)


# speedup vs baseline: 1.6909x; 1.2667x over previous
import jax
import jax.numpy as jnp
from jax import lax
from jax.experimental import pallas as pl
from jax.experimental.pallas import tpu as pltpu
from jax.experimental.pallas import tpu_sc as plsc

N = 10000
E = 320000
D = 128
DE = 16

NC = 2
NS = 16
NW = NC * NS
C = 64
NCHUNKS = E // C
NPAD = 10240

RB = 2000
NBB = E // RB
NAB = N // RB


NH = 5120
NHP = 5248
TRASH = 5200
ZROWS = NHP // NS
DROWS = NH // NS


def _remap(dst_v, idxp_v, lo):
    for u in range(C // 16):
        su = pl.ds(u * 16, 16)
        t = dst_v[su] - lo
        ok = (t >= 0) & (t < NH)
        idxp_v[su] = jnp.where(ok, t, TRASH)


def _zero_acc(zeros_hbm, agg_sp, sid):
    r0 = sid * ZROWS
    for d in range(ZROWS // C):
        pltpu.sync_copy(zeros_hbm, agg_sp.at[pl.ds(r0 + d * C, C)])
    rem = ZROWS % C
    if rem:
        pltpu.sync_copy(zeros_hbm.at[pl.ds(0, rem)],
                        agg_sp.at[pl.ds(r0 + (ZROWS // C) * C, rem)])


def _dump_acc(agg_sp, agg_hbm, cid, sid, out_base):
    for d in range(DROWS // C):
        r0 = sid * DROWS + d * C
        pltpu.sync_copy(agg_sp.at[pl.ds(r0, C)],
                        agg_hbm.at[cid, pl.ds(out_base + r0, C)])


def _worker_slab(wid):
    npair = jnp.where(wid < 4, 79, 78)
    base0 = 2 * C * (78 * wid + jnp.minimum(wid, 4))
    return npair, base0


def _edge_sc1(y_hbm, z_hbm, src_hbm, dst_hbm, zeros_hbm, msg_hbm, agg_hbm,
              src0, src1, dst0, dst1, ixp0, ixp1, yg0, yg1, z0, z1, m0, m1,
              agg_sp,
              ss0, ss1, sd0, sd1, sz0, sz1, sg0, sg1, sw0, sw1, sc0, sc1):
    cid = lax.axis_index("c")
    sid = lax.axis_index("s")
    wid = sid * NC + cid
    src_v = (src0, src1)
    dst_v = (dst0, dst1)
    ixp_v = (ixp0, ixp1)
    yg_v = (yg0, yg1)
    z_v = (z0, z1)
    m_v = (m0, m1)
    s_src = (ss0, ss1)
    s_dst = (sd0, sd1)
    s_z = (sz0, sz1)
    s_g = (sg0, sg1)
    s_w = (sw0, sw1)
    s_sc = (sc0, sc1)

    npair, base0 = _worker_slab(wid)

    def ebase(k):
        return base0 + k * C

    def zslice(k):
        return z_hbm.at[pl.ds(pl.multiple_of(ebase(k) // 8, 8), C // 8)]

    def issue_pref(k, b):
        pltpu.async_copy(src_hbm.at[pl.ds(ebase(k), C)], src_v[b], s_src[b])
        pltpu.async_copy(dst_hbm.at[pl.ds(ebase(k), C)], dst_v[b], s_dst[b])
        pltpu.async_copy(zslice(k), z_v[b], s_z[b])

    def wait_pref(k, b):
        pltpu.make_async_copy(src_hbm.at[pl.ds(ebase(k), C)], src_v[b],
                              s_src[b]).wait()
        pltpu.make_async_copy(dst_hbm.at[pl.ds(ebase(k), C)], dst_v[b],
                              s_dst[b]).wait()
        pltpu.make_async_copy(zslice(k), z_v[b], s_z[b]).wait()

    def issue_gather(k, b):
        pltpu.async_copy(y_hbm.at[src_v[b]], yg_v[b], s_g[b])

    def wait_gather(b):
        pltpu.make_async_copy(y_hbm.at[src_v[b]], yg_v[b], s_g[b]).wait()

    def compute(b):
        @pl.loop(0, C)
        def _row(r):
            zr = r // 8
            zc = (r % 8) * D
            for j in range(D // 16):
                s = pl.ds(j * 16, 16)
                m_v[b][r, s] = jnp.maximum(
                    yg_v[b][r, s] + z_v[b][zr, pl.ds(zc + j * 16, 16)], 0.0)

    _zero_acc(zeros_hbm, agg_sp, sid)
    plsc.subcore_barrier()

    issue_pref(0, 0)
    issue_pref(1, 1)
    wait_pref(0, 0)
    issue_gather(0, 0)

    @pl.loop(0, npair)
    def _pair(g):
        for b in (0, 1):
            k = 2 * g + b
            nb = 1 - b
            if b == 0:
                wait_pref(k + 1, nb)
                issue_gather(k + 1, nb)
            else:
                @pl.when(g < npair - 1)
                def _():
                    wait_pref(k + 1, nb)
                    issue_gather(k + 1, nb)
            wait_gather(b)

            @pl.when(g >= 1)
            def _():
                pltpu.make_async_copy(m_v[b],
                                      msg_hbm.at[pl.ds(ebase(k - 2), C)],
                                      s_w[b]).wait()
                pltpu.make_async_copy(m_v[b], agg_sp.at[ixp_v[b]],
                                      s_sc[b]).wait()

            compute(b)
            _remap(dst_v[b], ixp_v[b], 0)
            pltpu.async_copy(m_v[b], msg_hbm.at[pl.ds(ebase(k), C)], s_w[b])
            pltpu.async_copy(m_v[b], agg_sp.at[ixp_v[b]], s_sc[b], add=True)

            @pl.when(g < npair - 1)
            def _():
                issue_pref(k + 2, b)

    for b in (0, 1):
        pltpu.make_async_copy(m_v[b], msg_hbm.at[pl.ds(0, C)], s_w[b]).wait()
        pltpu.make_async_copy(m_v[b], agg_sp.at[ixp_v[b]], s_sc[b]).wait()

    plsc.subcore_barrier()
    _dump_acc(agg_sp, agg_hbm, cid, sid, 0)
    @pl.when(sid == 0)
    def _():
        pltpu.sync_copy(agg_sp.at[pl.ds(TRASH, 8)],
                        agg_hbm.at[cid, pl.ds(NH, 8)])


def _edge_sc2(dst_hbm, msg_hbm, zeros_hbm, agg_hbm,
              dst0, dst1, ixp0, ixp1, m0, m1, agg_sp,
              sd0, sd1, sw0, sw1, sc0, sc1):
    cid = lax.axis_index("c")
    sid = lax.axis_index("s")
    wid = sid * NC + cid
    dst_v = (dst0, dst1)
    ixp_v = (ixp0, ixp1)
    m_v = (m0, m1)
    s_dst = (sd0, sd1)
    s_w = (sw0, sw1)
    s_sc = (sc0, sc1)
    npair, base0 = _worker_slab(wid)

    def ebase(k):
        return base0 + k * C

    _zero_acc(zeros_hbm, agg_sp, sid)
    plsc.subcore_barrier()

    def issue_pref2(k, b):
        pltpu.async_copy(dst_hbm.at[pl.ds(ebase(k), C)], dst_v[b], s_dst[b])
        pltpu.async_copy(msg_hbm.at[pl.ds(ebase(k), C)], m_v[b], s_w[b])

    def wait_pref2(k, b):
        pltpu.make_async_copy(dst_hbm.at[pl.ds(ebase(k), C)], dst_v[b],
                              s_dst[b]).wait()
        pltpu.make_async_copy(msg_hbm.at[pl.ds(ebase(k), C)], m_v[b],
                              s_w[b]).wait()

    issue_pref2(0, 0)

    @pl.loop(0, npair)
    def _pair2(g):
        for b in (0, 1):
            k = 2 * g + b
            nb = 1 - b
            wait_pref2(k, b)
            _remap(dst_v[b], ixp_v[b], NH)
            pltpu.async_copy(m_v[b], agg_sp.at[ixp_v[b]], s_sc[b], add=True)
            if b == 0:
                @pl.when(g >= 1)
                def _():
                    pltpu.make_async_copy(m_v[nb], agg_sp.at[ixp_v[nb]],
                                          s_sc[nb]).wait()
                issue_pref2(k + 1, nb)
            else:
                @pl.when(g < npair - 1)
                def _():
                    pltpu.make_async_copy(m_v[nb], agg_sp.at[ixp_v[nb]],
                                          s_sc[nb]).wait()
                    issue_pref2(k + 1, nb)

    for b in (0, 1):
        pltpu.make_async_copy(m_v[b], agg_sp.at[ixp_v[b]], s_sc[b]).wait()

    plsc.subcore_barrier()
    _dump_acc(agg_sp, agg_hbm, cid, sid, 0)


def _edge_phase1(y, z, src_r, dst_r, zeros):
    mesh = plsc.VectorSubcoreMesh(core_axis_name="c", subcore_axis_name="s")
    return pl.kernel(
        _edge_sc1,
        out_type=(
            jax.ShapeDtypeStruct((E, D), jnp.float32),
            jax.ShapeDtypeStruct((NC, NH + 8, D), jnp.float32),
        ),
        mesh=mesh,
        scratch_types=(
            [pltpu.VMEM((C,), jnp.int32)] * 6 +
            [pltpu.VMEM((C, D), jnp.float32)] * 2 +
            [pltpu.VMEM((C // 8, 8 * D), jnp.float32)] * 2 +
            [pltpu.VMEM((C, D), jnp.float32)] * 2 +
            [pltpu.VMEM_SHARED((NHP, D), jnp.float32)] +
            [pltpu.SemaphoreType.DMA] * 12
        ),
    )(y, z, src_r, dst_r, zeros)


def _edge_phase2(dst_r, msg, zeros):
    mesh = plsc.VectorSubcoreMesh(core_axis_name="c", subcore_axis_name="s")
    return pl.kernel(
        _edge_sc2,
        out_type=jax.ShapeDtypeStruct((NC, NH, D), jnp.float32),
        mesh=mesh,
        scratch_types=(
            [pltpu.VMEM((C,), jnp.int32)] * 4 +
            [pltpu.VMEM((C, D), jnp.float32)] * 2 +
            [pltpu.VMEM_SHARED((NHP, D), jnp.float32)] +
            [pltpu.SemaphoreType.DMA] * 6
        ),
    )(dst_r, msg, zeros)


def _prep_y(x, W1):
    def body(x_ref, w_ref, y_ref):
        y_ref[...] = jnp.dot(x_ref[...], w_ref[...],
                             preferred_element_type=jnp.float32)

    return pl.pallas_call(
        body,
        out_shape=jax.ShapeDtypeStruct((N, D), jnp.float32),
    )(x, W1)


def _prep_z(ea8, W2b):
    def body(ea_ref, w_ref, z_ref):
        z_ref[...] = jnp.dot(ea_ref[...], w_ref[...],
                             preferred_element_type=jnp.float32)

    zrb = 400
    return pl.pallas_call(
        body,
        grid=(E // 8 // zrb,),
        in_specs=[
            pl.BlockSpec((zrb, 8 * DE), lambda i: (i, 0)),
            pl.BlockSpec((8 * DE, 8 * D), lambda i: (0, 0)),
        ],
        out_specs=pl.BlockSpec((zrb, 8 * D), lambda i: (i, 0)),
        out_shape=jax.ShapeDtypeStruct((E // 8, 8 * D), jnp.float32),
    )(ea8, W2b)


def _mvb_phase(agg_a, W_mol_b, W_ffn, b_ffn_2d):
    def body(agg_ref, wb_ref, wf_ref, bf_ref, mvb_ref, cb_ref):
        cs = (jnp.sum(agg_ref[0], axis=0, keepdims=True) +
              jnp.sum(agg_ref[1], axis=0, keepdims=True))
        mvb = jnp.dot(cs * (1.0 / E), wb_ref[...],
                      preferred_element_type=jnp.float32)
        mvb_ref[...] = mvb
        cb_ref[...] = jnp.dot(mvb, wf_ref[D:, :],
                              preferred_element_type=jnp.float32) + bf_ref[...]

    return pl.pallas_call(
        body,
        out_shape=(
            jax.ShapeDtypeStruct((1, D), jnp.float32),
            jax.ShapeDtypeStruct((1, 1), jnp.float32),
        ),
    )(agg_a, W_mol_b, W_ffn, b_ffn_2d)


def _fill_bonds(msg, mvb, cb, wt2):
    def body(msg_ref, mvb_ref, cb_ref, wt2_ref, v_ref, r_ref, t_ref):
        blk = msg_ref[...]
        v_ref[:, :D] = blk
        v_ref[:, D:] = jnp.broadcast_to(mvb_ref[...], (RB, D))
        rt = lax.dot_general(wt2_ref[...], blk, (((1,), (1,)), ((), ())),
                             preferred_element_type=jnp.float32)
        r_ref[...] = rt[0:1].reshape(1, 1, RB) + cb_ref[...]
        t_ref[...] = rt[1:2].reshape(1, 1, RB)

    return pl.pallas_call(
        body,
        grid=(NBB,),
        in_specs=[
            pl.BlockSpec((RB, D), lambda i: (i, 0)),
            pl.BlockSpec((1, D), lambda i: (0, 0)),
            pl.BlockSpec((1, 1), lambda i: (0, 0)),
            pl.BlockSpec((2, D), lambda i: (0, 0)),
        ],
        out_specs=[
            pl.BlockSpec((RB, 2 * D), lambda i: (i, 0)),
            pl.BlockSpec((1, 1, RB), lambda i: (i, 0, 0)),
            pl.BlockSpec((1, 1, RB), lambda i: (i, 0, 0)),
        ],
        out_shape=[
            jax.ShapeDtypeStruct((E + N, 2 * D), jnp.float32),
            jax.ShapeDtypeStruct((NBB, 1, RB), jnp.float32),
            jax.ShapeDtypeStruct((NBB, 1, RB), jnp.float32),
        ],
    )(msg, mvb, cb, wt2)


def _atoms_phase(agg_a, agg_b, x, W_node, W_mol_a, W_ffn, b_ffn_2d):
    def body(agg_a_ref, agg_b_ref, x_ref, wn_ref, wa_ref, wf_ref,
             bf_ref, atoms_ref, mva_ref, ca_ref):
        lo = agg_a_ref[0, :NH, :] + agg_a_ref[1, :NH, :]
        hi = agg_b_ref[0, :N - NH, :] + agg_b_ref[1, :N - NH, :]
        agg = jnp.concatenate([lo, hi], axis=0)
        pre = jnp.dot(agg, wn_ref[...],
                      preferred_element_type=jnp.float32) + x_ref[...]
        atoms = jnp.maximum(pre, 0.0)
        atoms_ref[...] = atoms
        mean_a = jnp.sum(atoms, axis=0, keepdims=True) * (1.0 / N)
        mva = jnp.dot(mean_a, wa_ref[...], preferred_element_type=jnp.float32)
        mva_ref[...] = mva
        ca_ref[...] = jnp.dot(mva, wf_ref[D:, :],
                              preferred_element_type=jnp.float32) + bf_ref[...]

    return pl.pallas_call(
        body,
        out_shape=(
            jax.ShapeDtypeStruct((N, D), jnp.float32),
            jax.ShapeDtypeStruct((1, D), jnp.float32),
            jax.ShapeDtypeStruct((1, 1), jnp.float32),
        ),
    )(agg_a, agg_b, x, W_node, W_mol_a, W_ffn, b_ffn_2d)


def _fill_atoms(v_partial, atoms_v, mva, ca, wt2):
    def body(v_in_ref, at_ref, mva_ref, ca_ref, wt2_ref, v_ref, r_ref, t_ref):
        blk = at_ref[...]
        v_ref[:, :D] = blk
        v_ref[:, D:] = jnp.broadcast_to(mva_ref[...], (RB, D))
        rt = lax.dot_general(wt2_ref[...], blk, (((1,), (1,)), ((), ())),
                             preferred_element_type=jnp.float32)
        r_ref[...] = rt[0:1].reshape(1, 1, RB) + ca_ref[...]
        t_ref[...] = rt[1:2].reshape(1, 1, RB)

    return pl.pallas_call(
        body,
        grid=(NAB,),
        in_specs=[
            pl.BlockSpec((RB, 2 * D), lambda i: (NBB + i, 0)),
            pl.BlockSpec((RB, D), lambda i: (i, 0)),
            pl.BlockSpec((1, D), lambda i: (0, 0)),
            pl.BlockSpec((1, 1), lambda i: (0, 0)),
            pl.BlockSpec((2, D), lambda i: (0, 0)),
        ],
        out_specs=[
            pl.BlockSpec((RB, 2 * D), lambda i: (NBB + i, 0)),
            pl.BlockSpec((1, 1, RB), lambda i: (i, 0, 0)),
            pl.BlockSpec((1, 1, RB), lambda i: (i, 0, 0)),
        ],
        out_shape=[
            jax.ShapeDtypeStruct((E + N, 2 * D), jnp.float32),
            jax.ShapeDtypeStruct((NAB, 1, RB), jnp.float32),
            jax.ShapeDtypeStruct((NAB, 1, RB), jnp.float32),
        ],
        input_output_aliases={0: 0},
    )(v_partial, atoms_v, mva, ca, wt2)


def kernel(x, edge_index, edge_attr, W_msg, W_node, W_mol_a, W_mol_b, W_ffn,
           b_ffn):
    x = x.astype(jnp.float32)
    ei = edge_index.astype(jnp.int32)
    src_r = ei[0]
    dst_r = ei[1]
    W1 = W_msg[:D]
    W2 = W_msg[D:]
    y = _prep_y(x, W1)
    ea8 = edge_attr.reshape(E // 8, 8 * DE)
    W2b = jnp.kron(jnp.eye(8, dtype=jnp.float32), W2)
    z = _prep_z(ea8, W2b)
    zeros = jnp.zeros((C, D), jnp.float32)
    b2 = jnp.reshape(b_ffn, (1, 1))
    wt2 = jnp.concatenate([W_ffn[:D].T, jnp.ones((1, D), jnp.float32)],
                          axis=0)
    msg, agg_a = _edge_phase1(y, z, src_r, dst_r, zeros)
    agg_b = _edge_phase2(dst_r, msg, zeros)
    mvb, cb = _mvb_phase(agg_a, W_mol_b, W_ffn, b2)
    v_partial, rb, tb = _fill_bonds(msg, mvb, cb, wt2)
    atoms_v, mva, ca = _atoms_phase(agg_a, agg_b, x, W_node, W_mol_a,
                                    W_ffn, b2)
    v_all, ra, ta = _fill_atoms(v_partial, atoms_v, mva, ca, wt2)
    r_all = jnp.concatenate([rb.reshape(E, 1), ra.reshape(N, 1)], axis=0)
    t_all = jnp.concatenate([tb.reshape(E, 1), ta.reshape(N, 1)[1:]], axis=0)
    return (r_all, t_all, v_all)


# speedup vs baseline: 2.0481x; 1.2112x over previous
import jax
import jax.numpy as jnp
from jax import lax
from jax.experimental import pallas as pl
from jax.experimental.pallas import tpu as pltpu
from jax.experimental.pallas import tpu_sc as plsc

N = 10000
E = 320000
D = 128
DE = 16

NC = 2
NS = 16
NW = NC * NS
C = 64
NCHUNKS = E // C
NPAD = 10240

RB = 2000
NBB = E // RB
NAB = N // RB


NH = 5120
NHP = 5248
TRASH = 5200
ZROWS = NHP // NS
DROWS = NH // NS


def _remap(dst_v, idxp_v, lo):
    for u in range(C // 16):
        su = pl.ds(u * 16, 16)
        t = dst_v[su] - lo
        ok = (t >= 0) & (t < NH)
        idxp_v[su] = jnp.where(ok, t, TRASH)


def _zero_acc(zeros_hbm, agg_sp, sid):
    r0 = sid * ZROWS
    for d in range(ZROWS // C):
        pltpu.sync_copy(zeros_hbm, agg_sp.at[pl.ds(r0 + d * C, C)])
    rem = ZROWS % C
    if rem:
        pltpu.sync_copy(zeros_hbm.at[pl.ds(0, rem)],
                        agg_sp.at[pl.ds(r0 + (ZROWS // C) * C, rem)])


def _dump_acc(agg_sp, agg_hbm, cid, sid, out_base):
    for d in range(DROWS // C):
        r0 = sid * DROWS + d * C
        pltpu.sync_copy(agg_sp.at[pl.ds(r0, C)],
                        agg_hbm.at[cid, pl.ds(out_base + r0, C)])
    rem = DROWS % C
    if rem:
        r0 = sid * DROWS + (DROWS // C) * C
        pltpu.sync_copy(agg_sp.at[pl.ds(r0, rem)],
                        agg_hbm.at[cid, pl.ds(out_base + r0, rem)])


def _worker_slab(wid):
    npair = jnp.where(wid < 4, 79, 78)
    base0 = 2 * C * (78 * wid + jnp.minimum(wid, 4))
    return npair, base0


def _edge_sc1(y_hbm, z_hbm, src_hbm, dst_hbm, zeros_hbm, msg_hbm, agg_hbm,
              src0, src1, dst0, dst1, ixp0, ixp1, yg0, yg1, z0, z1, m0, m1,
              agg_sp,
              ss0, ss1, sd0, sd1, sz0, sz1, sg0, sg1, sw0, sw1, sc0, sc1):
    cid = lax.axis_index("c")
    sid = lax.axis_index("s")
    wid = sid * NC + cid
    src_v = (src0, src1)
    dst_v = (dst0, dst1)
    ixp_v = (ixp0, ixp1)
    yg_v = (yg0, yg1)
    z_v = (z0, z1)
    m_v = (m0, m1)
    s_src = (ss0, ss1)
    s_dst = (sd0, sd1)
    s_z = (sz0, sz1)
    s_g = (sg0, sg1)
    s_w = (sw0, sw1)
    s_sc = (sc0, sc1)

    npair, base0 = _worker_slab(wid)

    def ebase(k):
        return base0 + k * C

    def zslice(k):
        return z_hbm.at[pl.ds(pl.multiple_of(ebase(k) // 8, 8), C // 8)]

    def issue_pref(k, b):
        pltpu.async_copy(src_hbm.at[pl.ds(ebase(k), C)], src_v[b], s_src[b])
        pltpu.async_copy(dst_hbm.at[pl.ds(ebase(k), C)], dst_v[b], s_dst[b])
        pltpu.async_copy(zslice(k), z_v[b], s_z[b])

    def wait_pref(k, b):
        pltpu.make_async_copy(src_hbm.at[pl.ds(ebase(k), C)], src_v[b],
                              s_src[b]).wait()
        pltpu.make_async_copy(dst_hbm.at[pl.ds(ebase(k), C)], dst_v[b],
                              s_dst[b]).wait()
        pltpu.make_async_copy(zslice(k), z_v[b], s_z[b]).wait()

    def issue_gather(k, b):
        pltpu.async_copy(y_hbm.at[src_v[b]], yg_v[b], s_g[b])

    def wait_gather(b):
        pltpu.make_async_copy(y_hbm.at[src_v[b]], yg_v[b], s_g[b]).wait()

    def compute(b):
        @plsc.parallel_loop(0, C // 8)
        def _grp(q):
            r0 = q * 8
            for e in range(8):
                for j in range(D // 16):
                    s = pl.ds(j * 16, 16)
                    m_v[b][r0 + e, s] = jnp.maximum(
                        yg_v[b][r0 + e, s] +
                        z_v[b][q, pl.ds(e * D + j * 16, 16)], 0.0)

    _zero_acc(zeros_hbm, agg_sp, sid)
    plsc.subcore_barrier()

    issue_pref(0, 0)
    issue_pref(1, 1)
    wait_pref(0, 0)
    issue_gather(0, 0)

    @pl.loop(0, npair)
    def _pair(g):
        for b in (0, 1):
            k = 2 * g + b
            nb = 1 - b
            if b == 0:
                wait_pref(k + 1, nb)
                issue_gather(k + 1, nb)
            else:
                @pl.when(g < npair - 1)
                def _():
                    wait_pref(k + 1, nb)
                    issue_gather(k + 1, nb)
            wait_gather(b)

            @pl.when(g >= 1)
            def _():
                pltpu.make_async_copy(m_v[b],
                                      msg_hbm.at[pl.ds(ebase(k - 2), C)],
                                      s_w[b]).wait()
                pltpu.make_async_copy(m_v[b], agg_sp.at[ixp_v[b]],
                                      s_sc[b]).wait()

            compute(b)
            _remap(dst_v[b], ixp_v[b], 0)
            pltpu.async_copy(m_v[b], msg_hbm.at[pl.ds(ebase(k), C)], s_w[b])
            pltpu.async_copy(m_v[b], agg_sp.at[ixp_v[b]], s_sc[b], add=True)

            @pl.when(g < npair - 1)
            def _():
                issue_pref(k + 2, b)

    for b in (0, 1):
        pltpu.make_async_copy(m_v[b], msg_hbm.at[pl.ds(0, C)], s_w[b]).wait()
        pltpu.make_async_copy(m_v[b], agg_sp.at[ixp_v[b]], s_sc[b]).wait()

    plsc.subcore_barrier()
    _dump_acc(agg_sp, agg_hbm, cid, sid, 0)
    @pl.when(sid == 0)
    def _():
        pltpu.sync_copy(agg_sp.at[pl.ds(TRASH, 8)],
                        agg_hbm.at[cid, pl.ds(NH, 8)])


def _edge_sc2(dst_hbm, msg_hbm, zeros_hbm, agg_hbm,
              dst0, dst1, ixp0, ixp1, m0, m1, agg_sp,
              sd0, sd1, sw0, sw1, sc0, sc1):
    cid = lax.axis_index("c")
    sid = lax.axis_index("s")
    wid = sid * NC + cid
    dst_v = (dst0, dst1)
    ixp_v = (ixp0, ixp1)
    m_v = (m0, m1)
    s_dst = (sd0, sd1)
    s_w = (sw0, sw1)
    s_sc = (sc0, sc1)
    npair, base0 = _worker_slab(wid)

    def ebase(k):
        return base0 + k * C

    _zero_acc(zeros_hbm, agg_sp, sid)
    plsc.subcore_barrier()

    def issue_pref2(k, b):
        pltpu.async_copy(dst_hbm.at[pl.ds(ebase(k), C)], dst_v[b], s_dst[b])
        pltpu.async_copy(msg_hbm.at[pl.ds(ebase(k), C)], m_v[b], s_w[b])

    def wait_pref2(k, b):
        pltpu.make_async_copy(dst_hbm.at[pl.ds(ebase(k), C)], dst_v[b],
                              s_dst[b]).wait()
        pltpu.make_async_copy(msg_hbm.at[pl.ds(ebase(k), C)], m_v[b],
                              s_w[b]).wait()

    issue_pref2(0, 0)

    @pl.loop(0, npair)
    def _pair2(g):
        for b in (0, 1):
            k = 2 * g + b
            nb = 1 - b
            wait_pref2(k, b)
            _remap(dst_v[b], ixp_v[b], NH)
            pltpu.async_copy(m_v[b], agg_sp.at[ixp_v[b]], s_sc[b], add=True)
            if b == 0:
                @pl.when(g >= 1)
                def _():
                    pltpu.make_async_copy(m_v[nb], agg_sp.at[ixp_v[nb]],
                                          s_sc[nb]).wait()
                issue_pref2(k + 1, nb)
            else:
                @pl.when(g < npair - 1)
                def _():
                    pltpu.make_async_copy(m_v[nb], agg_sp.at[ixp_v[nb]],
                                          s_sc[nb]).wait()
                    issue_pref2(k + 1, nb)

    for b in (0, 1):
        pltpu.make_async_copy(m_v[b], agg_sp.at[ixp_v[b]], s_sc[b]).wait()

    plsc.subcore_barrier()
    _dump_acc(agg_sp, agg_hbm, cid, sid, 0)


def _edge_phase1(y, z, src_r, dst_r, zeros):
    mesh = plsc.VectorSubcoreMesh(core_axis_name="c", subcore_axis_name="s")
    return pl.kernel(
        _edge_sc1,
        out_type=(
            jax.ShapeDtypeStruct((E, D), jnp.float32),
            jax.ShapeDtypeStruct((NC, NH + 8, D), jnp.float32),
        ),
        mesh=mesh,
        scratch_types=(
            [pltpu.VMEM((C,), jnp.int32)] * 6 +
            [pltpu.VMEM((C, D), jnp.float32)] * 2 +
            [pltpu.VMEM((C // 8, 8 * D), jnp.float32)] * 2 +
            [pltpu.VMEM((C, D), jnp.float32)] * 2 +
            [pltpu.VMEM_SHARED((NHP, D), jnp.float32)] +
            [pltpu.SemaphoreType.DMA] * 12
        ),
    )(y, z, src_r, dst_r, zeros)


def _edge_phase2(dst_r, msg, zeros):
    mesh = plsc.VectorSubcoreMesh(core_axis_name="c", subcore_axis_name="s")
    return pl.kernel(
        _edge_sc2,
        out_type=jax.ShapeDtypeStruct((NC, NH, D), jnp.float32),
        mesh=mesh,
        scratch_types=(
            [pltpu.VMEM((C,), jnp.int32)] * 4 +
            [pltpu.VMEM((C, D), jnp.float32)] * 2 +
            [pltpu.VMEM_SHARED((NHP, D), jnp.float32)] +
            [pltpu.SemaphoreType.DMA] * 6
        ),
    )(dst_r, msg, zeros)


def _prep_y(x, W1):
    def body(x_ref, w_ref, y_ref):
        y_ref[...] = jnp.dot(x_ref[...], w_ref[...],
                             preferred_element_type=jnp.float32)

    return pl.pallas_call(
        body,
        out_shape=jax.ShapeDtypeStruct((N, D), jnp.float32),
    )(x, W1)


def _prep_z(ea8, W2b):
    def body(ea_ref, w_ref, z_ref):
        z_ref[...] = jnp.dot(ea_ref[...], w_ref[...],
                             preferred_element_type=jnp.float32)

    zrb = 400
    return pl.pallas_call(
        body,
        grid=(E // 8 // zrb,),
        in_specs=[
            pl.BlockSpec((zrb, 8 * DE), lambda i: (i, 0)),
            pl.BlockSpec((8 * DE, 8 * D), lambda i: (0, 0)),
        ],
        out_specs=pl.BlockSpec((zrb, 8 * D), lambda i: (i, 0)),
        out_shape=jax.ShapeDtypeStruct((E // 8, 8 * D), jnp.float32),
    )(ea8, W2b)


def _mvb_phase(agg_a, W_mol_b, W_ffn, b_ffn_2d):
    def body(agg_ref, wb_ref, wf_ref, bf_ref, mvb_ref, cb_ref):
        cs = (jnp.sum(agg_ref[0], axis=0, keepdims=True) +
              jnp.sum(agg_ref[1], axis=0, keepdims=True))
        mvb = jnp.dot(cs * (1.0 / E), wb_ref[...],
                      preferred_element_type=jnp.float32)
        mvb_ref[...] = mvb
        cb_ref[...] = jnp.dot(mvb, wf_ref[D:, :],
                              preferred_element_type=jnp.float32) + bf_ref[...]

    return pl.pallas_call(
        body,
        out_shape=(
            jax.ShapeDtypeStruct((1, D), jnp.float32),
            jax.ShapeDtypeStruct((1, 1), jnp.float32),
        ),
    )(agg_a, W_mol_b, W_ffn, b_ffn_2d)


def _fill_bonds(msg, mvb, cb, wt2):
    def body(msg_ref, mvb_ref, cb_ref, wt2_ref, v_ref, r_ref, t_ref):
        blk = msg_ref[...]
        v_ref[:, :D] = blk
        v_ref[:, D:] = jnp.broadcast_to(mvb_ref[...], (RB, D))
        rt = lax.dot_general(wt2_ref[...], blk, (((1,), (1,)), ((), ())),
                             preferred_element_type=jnp.float32)
        r_ref[...] = rt[0:1].reshape(1, 1, RB) + cb_ref[...]
        t_ref[...] = rt[1:2].reshape(1, 1, RB)

    return pl.pallas_call(
        body,
        grid=(NBB,),
        in_specs=[
            pl.BlockSpec((RB, D), lambda i: (i, 0)),
            pl.BlockSpec((1, D), lambda i: (0, 0)),
            pl.BlockSpec((1, 1), lambda i: (0, 0)),
            pl.BlockSpec((2, D), lambda i: (0, 0)),
        ],
        out_specs=[
            pl.BlockSpec((RB, 2 * D), lambda i: (i, 0)),
            pl.BlockSpec((1, 1, RB), lambda i: (i, 0, 0)),
            pl.BlockSpec((1, 1, RB), lambda i: (i, 0, 0)),
        ],
        out_shape=[
            jax.ShapeDtypeStruct((E + N, 2 * D), jnp.float32),
            jax.ShapeDtypeStruct((NBB, 1, RB), jnp.float32),
            jax.ShapeDtypeStruct((NBB, 1, RB), jnp.float32),
        ],
    )(msg, mvb, cb, wt2)


def _atoms_phase(agg_a, agg_b, x, W_node, W_mol_a, W_ffn, b_ffn_2d):
    def body(agg_a_ref, agg_b_ref, x_ref, wn_ref, wa_ref, wf_ref,
             bf_ref, atoms_ref, mva_ref, ca_ref):
        lo = agg_a_ref[0, :NH, :] + agg_a_ref[1, :NH, :]
        hi = agg_b_ref[0, :N - NH, :] + agg_b_ref[1, :N - NH, :]
        agg = jnp.concatenate([lo, hi], axis=0)
        pre = jnp.dot(agg, wn_ref[...],
                      preferred_element_type=jnp.float32) + x_ref[...]
        atoms = jnp.maximum(pre, 0.0)
        atoms_ref[...] = atoms
        mean_a = jnp.sum(atoms, axis=0, keepdims=True) * (1.0 / N)
        mva = jnp.dot(mean_a, wa_ref[...], preferred_element_type=jnp.float32)
        mva_ref[...] = mva
        ca_ref[...] = jnp.dot(mva, wf_ref[D:, :],
                              preferred_element_type=jnp.float32) + bf_ref[...]

    return pl.pallas_call(
        body,
        out_shape=(
            jax.ShapeDtypeStruct((N, D), jnp.float32),
            jax.ShapeDtypeStruct((1, D), jnp.float32),
            jax.ShapeDtypeStruct((1, 1), jnp.float32),
        ),
    )(agg_a, agg_b, x, W_node, W_mol_a, W_ffn, b_ffn_2d)


def _fill_atoms(v_partial, atoms_v, mva, ca, wt2):
    def body(v_in_ref, at_ref, mva_ref, ca_ref, wt2_ref, v_ref, r_ref, t_ref):
        blk = at_ref[...]
        v_ref[:, :D] = blk
        v_ref[:, D:] = jnp.broadcast_to(mva_ref[...], (RB, D))
        rt = lax.dot_general(wt2_ref[...], blk, (((1,), (1,)), ((), ())),
                             preferred_element_type=jnp.float32)
        r_ref[...] = rt[0:1].reshape(1, 1, RB) + ca_ref[...]
        t_ref[...] = rt[1:2].reshape(1, 1, RB)

    return pl.pallas_call(
        body,
        grid=(NAB,),
        in_specs=[
            pl.BlockSpec((RB, 2 * D), lambda i: (NBB + i, 0)),
            pl.BlockSpec((RB, D), lambda i: (i, 0)),
            pl.BlockSpec((1, D), lambda i: (0, 0)),
            pl.BlockSpec((1, 1), lambda i: (0, 0)),
            pl.BlockSpec((2, D), lambda i: (0, 0)),
        ],
        out_specs=[
            pl.BlockSpec((RB, 2 * D), lambda i: (NBB + i, 0)),
            pl.BlockSpec((1, 1, RB), lambda i: (i, 0, 0)),
            pl.BlockSpec((1, 1, RB), lambda i: (i, 0, 0)),
        ],
        out_shape=[
            jax.ShapeDtypeStruct((E + N, 2 * D), jnp.float32),
            jax.ShapeDtypeStruct((NAB, 1, RB), jnp.float32),
            jax.ShapeDtypeStruct((NAB, 1, RB), jnp.float32),
        ],
        input_output_aliases={0: 0},
    )(v_partial, atoms_v, mva, ca, wt2)


def kernel(x, edge_index, edge_attr, W_msg, W_node, W_mol_a, W_mol_b, W_ffn,
           b_ffn):
    x = x.astype(jnp.float32)
    ei = edge_index.astype(jnp.int32)
    src_r = ei[0]
    dst_r = ei[1]
    W1 = W_msg[:D]
    W2 = W_msg[D:]
    y = _prep_y(x, W1)
    ea8 = edge_attr.reshape(E // 8, 8 * DE)
    W2b = jnp.kron(jnp.eye(8, dtype=jnp.float32), W2)
    z = _prep_z(ea8, W2b)
    zeros = jnp.zeros((C, D), jnp.float32)
    b2 = jnp.reshape(b_ffn, (1, 1))
    wt2 = jnp.concatenate([W_ffn[:D].T, jnp.ones((1, D), jnp.float32)],
                          axis=0)
    msg, agg_a = _edge_phase1(y, z, src_r, dst_r, zeros)
    agg_b = _edge_phase2(dst_r, msg, zeros)
    mvb, cb = _mvb_phase(agg_a, W_mol_b, W_ffn, b2)
    v_partial, rb, tb = _fill_bonds(msg, mvb, cb, wt2)
    atoms_v, mva, ca = _atoms_phase(agg_a, agg_b, x, W_node, W_mol_a,
                                    W_ffn, b2)
    v_all, ra, ta = _fill_atoms(v_partial, atoms_v, mva, ca, wt2)
    r_all = jnp.concatenate([rb.reshape(E, 1), ra.reshape(N, 1)], axis=0)
    t_all = jnp.concatenate([tb.reshape(E, 1), ta.reshape(N, 1)[1:]], axis=0)
    return (r_all, t_all, v_all)
